# Initial kernel scaffold; baseline (speedup 1.0000x reference)
import jax
import jax.numpy as jnp
from jax.experimental import pallas as pl


def kernel(video_features, query_embedding, mu_w1, mu_b1, mu_g1, mu_be1, mu_w2, mu_b2, sg_w1, sg_b1, sg_g1, sg_be1, sg_w2, sg_b2, in_proj_w, in_proj_b, out_w, out_b):
    raise NotImplementedError("write your pallas kernel here")


# speedup vs baseline: 1.7107x; 1.7107x over previous
import functools
import math

import jax
import jax.numpy as jnp
from jax import lax
from jax.experimental import pallas as pl
from jax.experimental.pallas import tpu as pltpu

D_M = 1024
K_REP = 6
TEMP = 0.1
N_HEADS = 8
HEAD_DIM = D_M // N_HEADS


def _query_stage_kernel(qe, mu_w1, mu_b1, mu_g1, mu_be1, mu_w2, mu_b2,
                        sg_w1, sg_b1, sg_g1, sg_be1, sg_w2, sg_b2,
                        wq, bq, wk,
                        mu_out, isg_out, u_out):
    def dot_t(x, w):
        return lax.dot_general(x, w, (((1,), (1,)), ((), ())),
                               preferred_element_type=jnp.float32)

    def layernorm(x, g, b):
        m = jnp.mean(x, axis=-1, keepdims=True)
        v = jnp.mean((x - m) ** 2, axis=-1, keepdims=True)
        return (x - m) / jnp.sqrt(v + 1e-5) * g + b

    def mlp(x, w1, b1, g, be, w2, b2):
        h = dot_t(x, w1) + b1[None, :]
        h = jnp.maximum(layernorm(h, g[None, :], be[None, :]), 0.0)
        return dot_t(h, w2) + b2[None, :]

    x = qe[...]
    mu = mlp(x, mu_w1[...], mu_b1[...], mu_g1[...], mu_be1[...],
             mu_w2[...], mu_b2[...])
    sg_pre = mlp(x, sg_w1[...], sg_b1[...], sg_g1[...], sg_be1[...],
                 sg_w2[...], sg_b2[...])
    sigma = jnp.maximum(sg_pre, 0.0) + jnp.log1p(jnp.exp(-jnp.abs(sg_pre)))
    sigma = sigma + 1e-6
    mu_out[...] = mu
    isg_out[...] = 1.0 / sigma

    q = dot_t(mu, wq[...]) + bq[...][None, :]
    bsz = q.shape[0]
    nrow = bsz * N_HEADS
    qb = jnp.broadcast_to(q[:, None, :], (bsz, N_HEADS, D_M)).reshape(nrow, D_M)
    head_of_col = lax.broadcasted_iota(jnp.int32, (nrow, D_M), 1) // HEAD_DIM
    head_of_row = lax.broadcasted_iota(jnp.int32, (nrow, D_M), 0) % N_HEADS
    q8 = jnp.where(head_of_col == head_of_row, qb, 0.0)
    scale = 1.0 / math.sqrt(HEAD_DIM)
    u = jnp.dot(q8, wk[...], preferred_element_type=jnp.float32) * scale
    u_out[...] = u.reshape(bsz, N_HEADS, D_M)


def _stream_stage_kernel(f_ref, mu_ref, isg_ref, u_ref, dist_ref, hl_ref):
    B = f_ref.shape[0]
    for b in range(B):
        f = f_ref[b]
        cen = f - mu_ref[b][None, :]
        dist_ref[b, 0, :] = jnp.sum(cen * cen * isg_ref[b][None, :], axis=1)
        hl = lax.dot_general(u_ref[b], f, (((1,), (1,)), ((), ())),
                             preferred_element_type=jnp.float32)
        hl_ref[b] = hl


def _finalize_kernel(dist_ref, hl_ref, mu_ref, isg_ref, feats_hbm,
                     rep_ref, idx_ref, comb_ref, loss_ref,
                     gather_vmem, sem):
    B, _, T = dist_ref.shape
    pos = lax.broadcasted_iota(jnp.int32, (B, T), 1)
    posf = pos.astype(jnp.float32)

    dist = dist_ref[:, 0, :]

    bits = lax.bitcast_convert_type(dist, jnp.int32)
    bp = bits.reshape(B * T // 128, 128)
    rank = (T - 1) // 2
    rows = T // 128

    med_s = []
    for b in range(B):
        bpb = bp[b * rows:(b + 1) * rows]

        def count_lt(cand):
            return jnp.sum((bpb < cand).astype(jnp.int32))

        m = jnp.int32(0)
        b30 = jnp.int32(1 << 30)
        m = jnp.where(count_lt(m | b30) <= rank, m | b30, m)
        for k in range(29, 0, -2):
            hi = jnp.int32(1 << k)
            lo = jnp.int32(1 << (k - 1))
            c_lo = count_lt(m | lo) <= rank
            c_hi = count_lt(m | hi) <= rank
            c_both = count_lt(m | hi | lo) <= rank
            m = m | jnp.where(c_hi, hi, jnp.int32(0))
            m = m | jnp.where(jnp.where(c_hi, c_both, c_lo), lo, jnp.int32(0))
        med_s.append(lax.bitcast_convert_type(m, jnp.float32))
    row_id = lax.broadcasted_iota(jnp.int32, (B, 1), 0)
    med = jnp.where(row_id == 0, med_s[0], med_s[1])

    dw = jnp.exp(-jnp.abs(dist - med) / TEMP)

    hl = hl_ref[...].reshape(B * N_HEADS, T)
    mx = jnp.max(hl, axis=1, keepdims=True)
    e = jnp.exp(hl - mx)
    attn = e / jnp.sum(e, axis=1, keepdims=True)
    attn_mean = jnp.mean(attn.reshape(B, N_HEADS, T), axis=1)

    w = dw * attn_mean
    comb_ref[...] = w

    def first_argmax(v):
        mv = jnp.max(v, axis=1, keepdims=True)
        return jnp.min(jnp.where(v == mv, pos, T), axis=1, keepdims=True)

    idxv = [first_argmax(w)]
    min_dist = jnp.abs(posf - idxv[0].astype(jnp.float32))
    sel = pos == idxv[0]
    for _ in range(K_REP - 1):
        score = jnp.where(sel, -jnp.inf, min_dist * w)
        nxt = first_argmax(score)
        idxv.append(nxt)
        min_dist = jnp.minimum(min_dist, jnp.abs(posf - nxt.astype(jnp.float32)))
        sel = sel | (pos == nxt)

    rd = [jnp.sum(jnp.where(pos == idxv[k], dist, 0.0), axis=1, keepdims=True)
          for k in range(K_REP)]
    t_rank = (K_REP - 1) // 2
    target = jnp.zeros((B, 1), jnp.float32)
    for i in range(K_REP):
        r_i = jnp.zeros((B, 1), jnp.int32)
        for j in range(K_REP):
            if j == i:
                continue
            less = rd[j] < rd[i]
            if j < i:
                less = less | (rd[j] == rd[i])
            r_i = r_i + less.astype(jnp.int32)
        target = target + jnp.where(r_i == t_rank, rd[i], 0.0)
    ell_sum = 0.0
    for k in range(K_REP):
        ell_sum = ell_sum + jnp.sum((rd[k] - target) ** 2)

    bsel = [lax.broadcasted_iota(jnp.int32, (B, 1), 0) == b for b in range(B)]
    con_sum = 0.0
    div_sum = 0.0
    for b in range(B):
        copies = []
        for k in range(K_REP):
            s = jnp.sum(jnp.where(bsel[b], idxv[k], 0))
            idx_ref[b, k] = s
            c = pltpu.make_async_copy(feats_hbm.at[b, s],
                                      gather_vmem.at[b, k], sem)
            c.start()
            copies.append(c)
        for c in copies:
            c.wait()
        rep = gather_vmem[b]
        rep_ref[b, :, :] = rep

        cen_rep = rep - mu_ref[b, :][None, :]
        con_sum = con_sum + jnp.sum(cen_rep * cen_rep * isg_ref[b, :][None, :])

        sim = lax.dot_general(rep, rep, (((1,), (1,)), ((), ())),
                              preferred_element_type=jnp.float32)
        ii = lax.broadcasted_iota(jnp.int32, (K_REP, K_REP), 0)
        jj = lax.broadcasted_iota(jnp.int32, (K_REP, K_REP), 1)
        upper = (jj > ii).astype(jnp.float32)
        div_sum = div_sum + jnp.sum((sim * upper) ** 2)

    ellipsoid = ell_sum / (B * K_REP)
    consistency = con_sum / (B * K_REP)
    diversity = div_sum / (B * K_REP * K_REP)
    loss_ref[0, 0] = ellipsoid + 0.1 * consistency + 0.05 * diversity


def kernel(video_features, query_embedding, mu_w1, mu_b1, mu_g1, mu_be1,
           mu_w2, mu_b2, sg_w1, sg_b1, sg_g1, sg_be1, sg_w2, sg_b2,
           in_proj_w, in_proj_b, out_w, out_b):
    B, T, D = video_features.shape
    wq = in_proj_w[:D]
    bq = in_proj_b[:D]
    wk = in_proj_w[D:2 * D]

    mu_q, inv_sigma, u = pl.pallas_call(
        _query_stage_kernel,
        out_shape=(
            jax.ShapeDtypeStruct((B, D), jnp.float32),
            jax.ShapeDtypeStruct((B, D), jnp.float32),
            jax.ShapeDtypeStruct((B, N_HEADS, D), jnp.float32),
        ),
    )(query_embedding, mu_w1, mu_b1, mu_g1, mu_be1, mu_w2, mu_b2,
      sg_w1, sg_b1, sg_g1, sg_be1, sg_w2, sg_b2, wq, bq, wk)

    TB = 512
    NT = T // TB
    dist, hlog = pl.pallas_call(
        _stream_stage_kernel,
        grid=(NT,),
        in_specs=[
            pl.BlockSpec((B, TB, D), lambda t: (0, t, 0)),
            pl.BlockSpec((B, D), lambda t: (0, 0)),
            pl.BlockSpec((B, D), lambda t: (0, 0)),
            pl.BlockSpec((B, N_HEADS, D), lambda t: (0, 0, 0)),
        ],
        out_specs=(
            pl.BlockSpec((B, 1, TB), lambda t: (0, 0, t)),
            pl.BlockSpec((B, N_HEADS, TB), lambda t: (0, 0, t)),
        ),
        out_shape=(
            jax.ShapeDtypeStruct((B, 1, T), jnp.float32),
            jax.ShapeDtypeStruct((B, N_HEADS, T), jnp.float32),
        ),
    )(video_features, mu_q, inv_sigma, u)

    rep, indices, combined, loss = pl.pallas_call(
        _finalize_kernel,
        in_specs=[
            pl.BlockSpec(memory_space=pl.ANY) if i == 4 else
            pl.BlockSpec(memory_space=pltpu.VMEM)
            for i in range(5)
        ],
        out_specs=(
            pl.BlockSpec(memory_space=pltpu.VMEM),
            pl.BlockSpec(memory_space=pltpu.SMEM),
            pl.BlockSpec(memory_space=pltpu.VMEM),
            pl.BlockSpec(memory_space=pltpu.SMEM),
        ),
        out_shape=(
            jax.ShapeDtypeStruct((B, K_REP, D), jnp.float32),
            jax.ShapeDtypeStruct((B, K_REP), jnp.int32),
            jax.ShapeDtypeStruct((B, T), jnp.float32),
            jax.ShapeDtypeStruct((1, 1), jnp.float32),
        ),
        scratch_shapes=[
            pltpu.VMEM((B, K_REP, D), jnp.float32),
            pltpu.SemaphoreType.DMA,
        ],
    )(dist, hlog, mu_q, inv_sigma, video_features)

    return rep, loss[0, 0], indices, combined


# speedup vs baseline: 2.2255x; 1.3009x over previous
import math

import jax
import jax.numpy as jnp
from jax import lax
from jax.experimental import pallas as pl
from jax.experimental.pallas import tpu as pltpu

D_M = 1024
K_REP = 6
TEMP = 0.1
N_HEADS = 8
HEAD_DIM = D_M // N_HEADS
TB = 512


def _dot_t(x, w):
    return lax.dot_general(x, w, (((1,), (1,)), ((), ())),
                           preferred_element_type=jnp.float32)


def _fused_kernel(f_blk, qe, mu_w1, mu_b1, mu_g1, mu_be1, mu_w2, mu_b2,
                  sg_w1, sg_b1, sg_g1, sg_be1, sg_w2, sg_b2,
                  wq, bq, wk, feats_hbm,
                  rep_ref, idx_ref, comb_ref, loss_ref,
                  mu_sc, isg_sc, u_sc, dist_sc, hl_sc, gather_vmem, sem):
    B = qe.shape[0]
    T = dist_sc.shape[1]
    NT = T // TB
    i = pl.program_id(0)

    @pl.when(i == 0)
    def _stage_q():
        def layernorm(x, g, b):
            m = jnp.mean(x, axis=-1, keepdims=True)
            v = jnp.mean((x - m) ** 2, axis=-1, keepdims=True)
            return (x - m) / jnp.sqrt(v + 1e-5) * g + b

        def mlp(x, w1, b1, g, be, w2, b2):
            h = _dot_t(x, w1[...]) + b1[...][None, :]
            h = jnp.maximum(layernorm(h, g[...][None, :], be[...][None, :]),
                            0.0)
            return _dot_t(h, w2[...]) + b2[...][None, :]

        x = qe[...]
        mu = mlp(x, mu_w1, mu_b1, mu_g1, mu_be1, mu_w2, mu_b2)
        sg_pre = mlp(x, sg_w1, sg_b1, sg_g1, sg_be1, sg_w2, sg_b2)
        sigma = jnp.maximum(sg_pre, 0.0) + jnp.log1p(jnp.exp(-jnp.abs(sg_pre)))
        sigma = sigma + 1e-6
        mu_sc[...] = mu
        isg_sc[...] = 1.0 / sigma

        q = _dot_t(mu, wq[...]) + bq[...][None, :]
        nrow = B * N_HEADS
        qb = jnp.broadcast_to(q[:, None, :], (B, N_HEADS, D_M)).reshape(
            nrow, D_M)
        col_h = lax.broadcasted_iota(jnp.int32, (nrow, D_M), 1) // HEAD_DIM
        row_h = lax.broadcasted_iota(jnp.int32, (nrow, D_M), 0) % N_HEADS
        q8 = jnp.where(col_h == row_h, qb, 0.0)
        scale = 1.0 / math.sqrt(HEAD_DIM)
        u_sc[...] = jnp.dot(q8, wk[...],
                            preferred_element_type=jnp.float32) * scale

    @pl.when((i >= 1) & (i <= NT))
    def _stage_stream():
        t0 = (i - 1) * TB
        for b in range(B):
            f = f_blk[b]
            cen = f - mu_sc[b][None, :]
            dist_sc[b, pl.ds(t0, TB)] = jnp.sum(
                cen * cen * isg_sc[b][None, :], axis=1)
            hl = lax.dot_general(
                u_sc[pl.ds(b * N_HEADS, N_HEADS), :], f,
                (((1,), (1,)), ((), ())),
                preferred_element_type=jnp.float32)
            hl_sc[pl.ds(b * N_HEADS, N_HEADS), pl.ds(t0, TB)] = hl

    @pl.when(i == NT + 1)
    def _stage_final():
        pos = lax.broadcasted_iota(jnp.int32, (B, T), 1)
        posf = pos.astype(jnp.float32)
        dist = dist_sc[...]

        bits = lax.bitcast_convert_type(dist, jnp.int32)
        bp = bits.reshape(B * T // 128, 128)
        rank = (T - 1) // 2
        rows = T // 128
        med_s = []
        for b in range(B):
            bpb = bp[b * rows:(b + 1) * rows]

            def count_lt(cand, _bpb=bpb):
                return jnp.sum((_bpb < cand).astype(jnp.int32))

            m = jnp.int32(0)
            b30 = jnp.int32(1 << 30)
            m = jnp.where(count_lt(m | b30) <= rank, m | b30, m)
            for k in range(29, 0, -2):
                hi = jnp.int32(1 << k)
                lo = jnp.int32(1 << (k - 1))
                c_lo = count_lt(m | lo) <= rank
                c_hi = count_lt(m | hi) <= rank
                c_both = count_lt(m | hi | lo) <= rank
                m = m | jnp.where(c_hi, hi, jnp.int32(0))
                m = m | jnp.where(jnp.where(c_hi, c_both, c_lo), lo,
                                  jnp.int32(0))
            med_s.append(lax.bitcast_convert_type(m, jnp.float32))
        row_id = lax.broadcasted_iota(jnp.int32, (B, 1), 0)
        med = jnp.where(row_id == 0, med_s[0], med_s[1])

        dw = jnp.exp(-jnp.abs(dist - med) / TEMP)

        hl = hl_sc[...]
        mx = jnp.max(hl, axis=1, keepdims=True)
        e = jnp.exp(hl - mx)
        attn = e / jnp.sum(e, axis=1, keepdims=True)
        attn_mean = jnp.mean(attn.reshape(B, N_HEADS, T), axis=1)

        w = dw * attn_mean
        comb_ref[...] = w

        def first_argmax(v):
            mv = jnp.max(v, axis=1, keepdims=True)
            return jnp.min(jnp.where(v == mv, pos, T), axis=1, keepdims=True)

        idxv = [first_argmax(w)]
        min_dist = jnp.abs(posf - idxv[0].astype(jnp.float32))
        sel = pos == idxv[0]
        for _ in range(K_REP - 1):
            score = jnp.where(sel, -jnp.inf, min_dist * w)
            nxt = first_argmax(score)
            idxv.append(nxt)
            min_dist = jnp.minimum(min_dist,
                                   jnp.abs(posf - nxt.astype(jnp.float32)))
            sel = sel | (pos == nxt)

        rd = [jnp.sum(jnp.where(pos == idxv[k], dist, 0.0), axis=1,
                      keepdims=True) for k in range(K_REP)]
        t_rank = (K_REP - 1) // 2
        target = jnp.zeros((B, 1), jnp.float32)
        for a in range(K_REP):
            r_a = jnp.zeros((B, 1), jnp.int32)
            for j in range(K_REP):
                if j == a:
                    continue
                less = rd[j] < rd[a]
                if j < a:
                    less = less | (rd[j] == rd[a])
                r_a = r_a + less.astype(jnp.int32)
            target = target + jnp.where(r_a == t_rank, rd[a], 0.0)
        ell_sum = 0.0
        for k in range(K_REP):
            ell_sum = ell_sum + jnp.sum((rd[k] - target) ** 2)

        bsel = [lax.broadcasted_iota(jnp.int32, (B, 1), 0) == b
                for b in range(B)]
        con_sum = 0.0
        div_sum = 0.0
        for b in range(B):
            copies = []
            for k in range(K_REP):
                s = jnp.sum(jnp.where(bsel[b], idxv[k], 0))
                idx_ref[b, k] = s
                c = pltpu.make_async_copy(feats_hbm.at[b, s],
                                          gather_vmem.at[b, k], sem)
                c.start()
                copies.append(c)
            for c in copies:
                c.wait()
            rep = gather_vmem[b]
            rep_ref[b, :, :] = rep

            cen_rep = rep - mu_sc[b][None, :]
            con_sum = con_sum + jnp.sum(cen_rep * cen_rep
                                        * isg_sc[b][None, :])
            sim = _dot_t(rep, rep)
            ii = lax.broadcasted_iota(jnp.int32, (K_REP, K_REP), 0)
            jj = lax.broadcasted_iota(jnp.int32, (K_REP, K_REP), 1)
            div_sum = div_sum + jnp.sum(
                (sim * (jj > ii).astype(jnp.float32)) ** 2)

        ellipsoid = ell_sum / (B * K_REP)
        consistency = con_sum / (B * K_REP)
        diversity = div_sum / (B * K_REP * K_REP)
        loss_ref[0, 0] = ellipsoid + 0.1 * consistency + 0.05 * diversity


def kernel(video_features, query_embedding, mu_w1, mu_b1, mu_g1, mu_be1,
           mu_w2, mu_b2, sg_w1, sg_b1, sg_g1, sg_be1, sg_w2, sg_b2,
           in_proj_w, in_proj_b, out_w, out_b):
    B, T, D = video_features.shape
    NT = T // TB

    def const2(_):
        return (0, 0)

    def const1(_):
        return (0,)

    full2 = pl.BlockSpec((D, D), const2)
    full1 = pl.BlockSpec((D,), const1)

    rep, indices, combined, loss = pl.pallas_call(
        _fused_kernel,
        grid=(NT + 2,),
        in_specs=[
            pl.BlockSpec((B, TB, D),
                         lambda i: (0, jnp.clip(i - 1, 0, NT - 1), 0)),
            pl.BlockSpec((B, D), const2),
            full2, full1, full1, full1, full2, full1,
            full2, full1, full1, full1, full2, full1,
            pl.BlockSpec((D, D), lambda i: (0, 0)),
            pl.BlockSpec((D,), lambda i: (0,)),
            pl.BlockSpec((D, D), lambda i: (1, 0)),
            pl.BlockSpec(memory_space=pl.ANY),
        ],
        out_specs=(
            pl.BlockSpec((B, K_REP, D), lambda i: (0, 0, 0)),
            pl.BlockSpec(memory_space=pltpu.SMEM),
            pl.BlockSpec((B, T), const2),
            pl.BlockSpec(memory_space=pltpu.SMEM),
        ),
        out_shape=(
            jax.ShapeDtypeStruct((B, K_REP, D), jnp.float32),
            jax.ShapeDtypeStruct((B, K_REP), jnp.int32),
            jax.ShapeDtypeStruct((B, T), jnp.float32),
            jax.ShapeDtypeStruct((1, 1), jnp.float32),
        ),
        scratch_shapes=[
            pltpu.VMEM((B, D), jnp.float32),
            pltpu.VMEM((B, D), jnp.float32),
            pltpu.VMEM((B * N_HEADS, D), jnp.float32),
            pltpu.VMEM((B, T), jnp.float32),
            pltpu.VMEM((B * N_HEADS, T), jnp.float32),
            pltpu.VMEM((B, K_REP, D), jnp.float32),
            pltpu.SemaphoreType.DMA,
        ],
    )(video_features, query_embedding, mu_w1, mu_b1, mu_g1, mu_be1,
      mu_w2, mu_b2, sg_w1, sg_b1, sg_g1, sg_be1, sg_w2, sg_b2,
      in_proj_w, in_proj_b, in_proj_w, video_features)

    return rep, loss[0, 0], indices, combined
